# baseline (speedup 1.0000x reference)
import functools

import jax
import jax.numpy as jnp
from jax import lax
from jax.experimental import pallas as pl
from jax.experimental.pallas import tpu as pltpu
from jax.experimental.pallas import tpu_sc as plsc

_G = 128
_K = 4


@functools.partial(jax.jit, static_argnums=(2, 3))
def _gather_sc(idx2d, table, B, D):
    info = plsc.get_sparse_core_info()
    NC, NS = info.num_cores, info.num_subcores
    NW = NC * NS
    per_w = B // NW
    S = _K * _G
    n_chunks = per_w // S
    rows_per_w = per_w // _G

    n_pairs = n_chunks // 2
    mesh = plsc.VectorSubcoreMesh(core_axis_name="c", subcore_axis_name="s")

    @functools.partial(
        pl.kernel,
        mesh=mesh,
        out_type=jax.ShapeDtypeStruct((B, D), jnp.float32),
        scratch_types=[
            pltpu.VMEM((2 * _K, _G), jnp.int32),
            pltpu.VMEM((S, D), jnp.float32),
            pltpu.VMEM((S, D), jnp.float32),
            pltpu.SemaphoreType.DMA,
            pltpu.SemaphoreType.DMA,
            pltpu.SemaphoreType.DMA,
        ],
        compiler_params=pltpu.CompilerParams(use_tc_tiling_on_sc=False),
    )
    def _k(idx_hbm, table_hbm, out_hbm, idx_v, rows0, rows1, sg0, sg1, sw):
        wid = lax.axis_index("s") * NC + lax.axis_index("c")
        row0 = wid * rows_per_w

        def body(p, carry):
            rbase = row0 + p * 2 * _K
            pltpu.sync_copy(idx_hbm.at[pl.ds(rbase, 2 * _K)], idx_v)
            g0 = [
                pltpu.async_copy(
                    table_hbm.at[idx_v.at[j]],
                    rows0.at[pl.ds(j * _G, _G)],
                    sg0,
                )
                for j in range(_K)
            ]
            g1 = [
                pltpu.async_copy(
                    table_hbm.at[idx_v.at[_K + j]],
                    rows1.at[pl.ds(j * _G, _G)],
                    sg1,
                )
                for j in range(_K)
            ]
            for cp in g0:
                cp.wait()
            w0 = pltpu.async_copy(rows0, out_hbm.at[pl.ds(rbase * _G, S)], sw)
            for cp in g1:
                cp.wait()
            w1 = pltpu.async_copy(
                rows1, out_hbm.at[pl.ds((rbase + _K) * _G, S)], sw
            )
            w0.wait()
            w1.wait()
            return carry

        lax.fori_loop(0, n_pairs, body, 0)

    return _k(idx2d, table)


def kernel(x, table):
    B = x.shape[0] * x.shape[1]
    D = table.shape[1]
    idx2d = x.reshape(B // _G, _G).astype(jnp.int32)
    out = _gather_sc(idx2d, table, B, D)
    return out.reshape(x.shape[0], x.shape[1], D)
